# Initial kernel scaffold; baseline (speedup 1.0000x reference)
import jax
import jax.numpy as jnp
from jax.experimental import pallas as pl


def kernel(x_pf, batch_pf, W_pf0, b_pf0, W_pf1, b_pf1, Wc1, bc1, Wc2, bc2, Wc3, bc3, Wo1, bo1, Wo2, bo2, Wo3, bo3, Wo4, bo4):
    raise NotImplementedError("write your pallas kernel here")


# speedup vs baseline: 9.1030x; 9.1030x over previous
import functools

import jax
import jax.numpy as jnp
from jax import lax
from jax.experimental import pallas as pl
from jax.experimental.pallas import tpu as pltpu
from jax.experimental.pallas import tpu_sc as plsc

N = 4096
DIN = 16
H = 256
KNN = 24
G = 32
RB = 128
CB = 128
NB = N // RB
INF = float("inf")
BIGF = 1e9


def _elu(v):
    return jnp.where(v > 0, v, jnp.exp(v) - 1.0)


def _init_body(x_ref, w0_ref, b0_ref, w1_ref, b1_ref, z_ref):
    h = _elu(jnp.dot(x_ref[...], w0_ref[...],
                     preferred_element_type=jnp.float32) + b0_ref[...])
    z_ref[...] = _elu(jnp.dot(h, w1_ref[...],
                              preferred_element_type=jnp.float32) + b1_ref[...])


def _input_mlp(x, w0, b0, w1, b1):
    return pl.pallas_call(
        _init_body,
        grid=(NB,),
        in_specs=[
            pl.BlockSpec((RB, DIN), lambda i: (i, 0)),
            pl.BlockSpec((DIN, H), lambda i: (0, 0)),
            pl.BlockSpec((1, H), lambda i: (0, 0)),
            pl.BlockSpec((H, H), lambda i: (0, 0)),
            pl.BlockSpec((1, H), lambda i: (0, 0)),
        ],
        out_specs=pl.BlockSpec((RB, H), lambda i: (i, 0)),
        out_shape=jax.ShapeDtypeStruct((N, H), jnp.float32),
    )(x, w0, b0, w1, b1)


def _knnab_body(f_full_ref, f_blk_ref, br_ref, bc_ref, wc_ref, bcb_ref,
                idx_ref, a_ref, bj_ref, d_ref):
    fr = f_blk_ref[...]
    wt = wc_ref[:H, :]
    wb = wc_ref[H:, :]
    a_ref[...] = jnp.dot(fr, wt - wb,
                         preferred_element_type=jnp.float32) + bcb_ref[...]
    bj_ref[...] = jnp.dot(fr, wb, preferred_element_type=jnp.float32)

    b_row = br_ref[...]
    bc_full = bc_ref[...]
    b_lo = br_ref[0, 0]
    b_hi = br_ref[RB - 1, 0]

    sq_r = jnp.sum(fr * fr, axis=1, keepdims=True)
    lo_cnt = jnp.sum((bc_full < b_lo).astype(jnp.float32)).astype(jnp.int32)
    hi_cnt = jnp.sum((bc_full <= b_hi).astype(jnp.float32)).astype(jnp.int32)
    c0 = (lo_cnt // CB) * CB
    nch = (hi_cnt - c0 + CB - 1) // CB

    def dist_body(j, carry):
        s_acc, e_acc = carry
        cstart = c0 + j * CB
        zc = f_full_ref[pl.ds(cstart, CB), :]
        bcch = bc_ref[:, pl.ds(cstart, CB)]
        dot = lax.dot_general(fr, zc, (((1,), (1,)), ((), ())),
                              preferred_element_type=jnp.float32)
        ones = jnp.ones((1, H), jnp.float32)
        sqc = lax.dot_general(ones, zc * zc, (((1,), (1,)), ((), ())),
                              preferred_element_type=jnp.float32)
        d2 = sq_r + sqc - 2.0 * dot
        same = bcch == b_row
        d_ref[:, pl.ds(j * CB, CB)] = jnp.where(same, d2, INF)
        s_acc = s_acc + jnp.sum((bcch < b_row).astype(jnp.float32),
                                axis=1, keepdims=True)
        e_acc = e_acc + jnp.sum((bcch <= b_row).astype(jnp.float32),
                                axis=1, keepdims=True)
        return s_acc, e_acc

    zero = jnp.zeros((RB, 1), jnp.float32)
    s_acc, e_acc = lax.fori_loop(0, nch, dist_body, (zero, zero))
    s_i = c0 + s_acc.astype(jnp.int32)
    n_i = (e_acc - s_acc).astype(jnp.int32)

    prevv = jnp.full((RB, 1), -INF, jnp.float32)
    previ = jnp.full((RB, 1), -1.0, jnp.float32)
    for t in range(KNN):
        def ext_body(j, carry, prevv=prevv, previ=previ):
            mval, marg = carry
            colw = (jax.lax.broadcasted_iota(jnp.int32, (1, CB), 1)
                    .astype(jnp.float32) + (j * CB).astype(jnp.float32))
            dch = d_ref[:, pl.ds(j * CB, CB)]
            avail = (dch > prevv) | ((dch == prevv) & (colw > previ))
            dm = jnp.where(avail, dch, INF)
            cmin = jnp.min(dm, axis=1, keepdims=True)
            carg = jnp.min(jnp.where(dm == cmin, colw, BIGF),
                           axis=1, keepdims=True)
            upd = cmin < mval
            return (jnp.where(upd, cmin, mval), jnp.where(upd, carg, marg))

        mval, marg = lax.fori_loop(
            0, nch, ext_body,
            (jnp.full((RB, 1), INF), jnp.full((RB, 1), 0.0, jnp.float32)))
        real = marg.astype(jnp.int32) + c0
        r = t - n_i
        junk = jnp.where(r < s_i, r, r + n_i)
        sel = n_i > t
        idx_ref[:, pl.ds(t, 1)] = jnp.where(sel, real, junk)
        prevv, previ = mval, marg

    _ = b_row


def _knn_ab(f, batch_r, batch_c, wc, bc):
    return pl.pallas_call(
        _knnab_body,
        grid=(NB,),
        in_specs=[
            pl.BlockSpec((N, H), lambda i: (0, 0)),
            pl.BlockSpec((RB, H), lambda i: (i, 0)),
            pl.BlockSpec((RB, 1), lambda i: (i, 0)),
            pl.BlockSpec((1, N), lambda i: (0, 0)),
            pl.BlockSpec((2 * H, H), lambda i: (0, 0)),
            pl.BlockSpec((1, H), lambda i: (0, 0)),
        ],
        out_specs=[
            pl.BlockSpec((RB, KNN), lambda i: (i, 0)),
            pl.BlockSpec((RB, H), lambda i: (i, 0)),
            pl.BlockSpec((RB, H), lambda i: (i, 0)),
        ],
        out_shape=[
            jax.ShapeDtypeStruct((N, KNN), jnp.int32),
            jax.ShapeDtypeStruct((N, H), jnp.float32),
            jax.ShapeDtypeStruct((N, H), jnp.float32),
        ],
        scratch_shapes=[pltpu.VMEM((RB, N), jnp.float32)],
    )(f, f, batch_r, batch_c, wc, bc)


SC_NODES_PER_CHUNK = 8
SC_ROWS = SC_NODES_PER_CHUNK * KNN


def _sc_gather_max(bj, idx_flat, a):
    info = plsc.get_sparse_core_info()
    nc, ns = info.num_cores, info.num_subcores
    nw = nc * ns
    nodes_per_w = N // nw
    nchunks = nodes_per_w // SC_NODES_PER_CHUNK
    mesh = plsc.VectorSubcoreMesh(core_axis_name="c", subcore_axis_name="s")

    @functools.partial(
        pl.kernel, mesh=mesh,
        out_type=jax.ShapeDtypeStruct((N, H), jnp.float32),
        scratch_types=[
            pltpu.VMEM((SC_ROWS,), jnp.int32),
            pltpu.VMEM((SC_ROWS,), jnp.int32),
            pltpu.VMEM((SC_ROWS, H), jnp.float32),
            pltpu.VMEM((SC_ROWS, H), jnp.float32),
            pltpu.VMEM((SC_NODES_PER_CHUNK, H), jnp.float32),
            pltpu.VMEM((SC_NODES_PER_CHUNK, H), jnp.float32),
            pltpu.SemaphoreType.DMA,
            pltpu.SemaphoreType.DMA,
        ],
    )
    def k(bj_hbm, idxf_hbm, a_hbm, out_hbm,
          idx0, idx1, rows0, rows1, a_v, out_v, sem0, sem1):
        wid = lax.axis_index("s") * nc + lax.axis_index("c")
        w0 = wid * nodes_per_w
        bufs = ((idx0, rows0, sem0), (idx1, rows1, sem1))

        def fire(c, b):
            idx_v, rows_v, sem = bufs[b]
            base = w0 + c * SC_NODES_PER_CHUNK
            pltpu.sync_copy(idxf_hbm.at[pl.ds(base * KNN, SC_ROWS)], idx_v)
            pltpu.async_copy(bj_hbm.at[idx_v], rows_v, sem)

        def drain_compute(c, b):
            idx_v, rows_v, sem = bufs[b]
            base = w0 + c * SC_NODES_PER_CHUNK
            pltpu.make_async_copy(bj_hbm.at[idx_v], rows_v, sem).wait()
            pltpu.sync_copy(a_hbm.at[pl.ds(base, SC_NODES_PER_CHUNK)], a_v)

            def node_body(n, _):
                r0 = n * KNN
                for q in range(H // 16):
                    sl = pl.ds(q * 16, 16)
                    acc = rows_v[r0, sl]
                    for j in range(1, KNN):
                        acc = jnp.maximum(acc, rows_v[r0 + j, sl])
                    v = a_v[n, sl] + acc
                    out_v[n, sl] = jnp.where(v > 0, v, jnp.exp(v) - 1.0)
                return 0

            lax.fori_loop(0, SC_NODES_PER_CHUNK, node_body, 0)
            pltpu.sync_copy(out_v, out_hbm.at[pl.ds(base, SC_NODES_PER_CHUNK)])

        fire(0, 0)

        def outer(g, _):
            for b in range(2):
                c = g * 2 + b

                @pl.when(c + 1 < nchunks)
                def _():
                    fire(c + 1, 1 - b)

                drain_compute(c, b)
            return 0

        lax.fori_loop(0, nchunks // 2, outer, 0)

    return k(bj, idx_flat, a)


def _tail_body(f_ref, bc_ref, wo1_ref, bo1_ref, wo2_ref, bo2_ref,
               wo3_ref, bo3_ref, wo4_ref, bo4_ref, out_ref, pool_ref):
    i = pl.program_id(0)

    @pl.when(i == 0)
    def _():
        pool_ref[...] = jnp.zeros((G, H), jnp.float32)

    bcch = bc_ref[:, pl.ds(i * RB, RB)]
    gidx = jax.lax.broadcasted_iota(jnp.int32, (G, RB), 0).astype(jnp.float32)
    oh = (gidx == bcch).astype(jnp.float32)
    pool_ref[...] += jnp.dot(oh, f_ref[...],
                             preferred_element_type=jnp.float32)

    @pl.when(i == NB - 1)
    def _():
        o = _elu(jnp.dot(pool_ref[...], wo1_ref[...],
                         preferred_element_type=jnp.float32) + bo1_ref[...])
        o = _elu(jnp.dot(o, wo2_ref[...],
                         preferred_element_type=jnp.float32) + bo2_ref[...])
        o = _elu(jnp.dot(o, wo3_ref[...],
                         preferred_element_type=jnp.float32) + bo3_ref[...])
        out_ref[...] = jnp.dot(o, wo4_ref[...],
                               preferred_element_type=jnp.float32) + bo4_ref[...]


def _tail(f3, batch_c, wo1, bo1, wo2, bo2, wo3, bo3, wo4, bo4):
    return pl.pallas_call(
        _tail_body,
        grid=(NB,),
        in_specs=[
            pl.BlockSpec((RB, H), lambda i: (i, 0)),
            pl.BlockSpec((1, N), lambda i: (0, 0)),
            pl.BlockSpec((H, 64), lambda i: (0, 0)),
            pl.BlockSpec((1, 64), lambda i: (0, 0)),
            pl.BlockSpec((64, 32), lambda i: (0, 0)),
            pl.BlockSpec((1, 32), lambda i: (0, 0)),
            pl.BlockSpec((32, 32), lambda i: (0, 0)),
            pl.BlockSpec((1, 32), lambda i: (0, 0)),
            pl.BlockSpec((32, 8), lambda i: (0, 0)),
            pl.BlockSpec((1, 8), lambda i: (0, 0)),
        ],
        out_specs=pl.BlockSpec((G, 8), lambda i: (0, 0)),
        out_shape=jax.ShapeDtypeStruct((G, 8), jnp.float32),
        scratch_shapes=[pltpu.VMEM((G, H), jnp.float32)],
    )(f3, batch_c, wo1, bo1, wo2, bo2, wo3, bo3, wo4, bo4)


def kernel(x_pf, batch_pf, W_pf0, b_pf0, W_pf1, b_pf1, Wc1, bc1, Wc2, bc2,
           Wc3, bc3, Wo1, bo1, Wo2, bo2, Wo3, bo3, Wo4, bo4):
    batch_f = batch_pf.astype(jnp.float32)
    batch_r = batch_f.reshape(N, 1)
    batch_c = batch_f.reshape(1, N)
    row = lambda b: b.reshape(1, -1)

    z = _input_mlp(x_pf, W_pf0, row(b_pf0), W_pf1, row(b_pf1))

    f = z
    for wc, bcv in ((Wc1, bc1), (Wc2, bc2), (Wc3, bc3)):
        idx, a, bj = _knn_ab(f, batch_r, batch_c, wc, row(bcv))
        f = _sc_gather_max(bj, idx.reshape(-1), a)

    out = _tail(f, batch_c, Wo1, row(bo1), Wo2, row(bo2),
                Wo3, row(bo3), Wo4, row(bo4))
    return (out, batch_pf)
